# baseline (device time: 6752 ns/iter reference)
import jax
import jax.numpy as jnp
from jax import lax
from jax.experimental import pallas as pl
from jax.experimental.pallas import tpu as pltpu


def kernel(x, dy, gamma):
    m, d = x.shape

    def body(x_ref, dy_ref, out_ref):
        xv = x_ref[:, :]
        dyv = dy_ref[:, :]
        inv_d = 1.0 / d
        s1 = jnp.sum(xv, axis=1, keepdims=True) * inv_d
        s2 = jnp.sum(xv * xv, axis=1, keepdims=True) * inv_d
        a = lax.rsqrt(s2 - s1 * s1 + 1e-5)
        b = s1 * a
        xhat = xv * a - b
        out_ref[0:1, :] = jnp.sum(dyv * xhat, axis=0, keepdims=True)
        out_ref[1:2, :] = jnp.sum(dyv, axis=0, keepdims=True)

    return pl.pallas_call(
        body,
        out_shape=jax.ShapeDtypeStruct((2, d), jnp.float32),
        in_specs=[
            pl.BlockSpec(memory_space=pltpu.VMEM),
            pl.BlockSpec(memory_space=pltpu.VMEM),
        ],
        out_specs=pl.BlockSpec(memory_space=pltpu.VMEM),
    )(x, dy)
